# baseline (device time: 214236 ns/iter reference)
import jax
import jax.numpy as jnp
from jax import lax
from jax.experimental import pallas as pl
from jax.experimental.pallas import tpu as pltpu

N_CHUNKS = 8


def kernel(x):
    m, n = x.shape
    half = n // 2
    mc = m // N_CHUNKS

    def body(x_ref, out_ref, in_buf, send_buf, loc_buf,
             in_sems, out_sems, send_sems, recv_sems):
        my_p = lax.axis_index("x")
        my_y = lax.axis_index("y")
        my_z = lax.axis_index("z")
        peer = 1 - my_p

        def in_dma(k):
            return pltpu.make_async_copy(
                x_ref.at[pl.ds(k * mc, mc), :],
                in_buf.at[k % 2],
                in_sems.at[k % 2],
            )

        def out_dma(k):
            return pltpu.make_async_copy(
                loc_buf.at[k % 2],
                out_ref.at[pl.ds(my_p * m + k * mc, mc), :],
                out_sems.at[k % 2],
            )

        in_dma(0).start()
        rdmas = []
        for k in range(N_CHUNKS):
            in_dma(k).wait()
            if k + 1 < N_CHUNKS:
                in_dma(k + 1).start()

            if k >= 2:
                out_dma(k - 2).wait()

            @pl.when(my_p == 0)
            def _():
                loc_buf[k % 2] = in_buf[k % 2, :, :half].astype(jnp.bfloat16)
                send_buf[k] = in_buf[k % 2, :, half:].astype(jnp.bfloat16)

            @pl.when(my_p == 1)
            def _():
                loc_buf[k % 2] = in_buf[k % 2, :, half:].astype(jnp.bfloat16)
                send_buf[k] = in_buf[k % 2, :, :half].astype(jnp.bfloat16)

            out_dma(k).start()
            r = pltpu.make_async_remote_copy(
                src_ref=send_buf.at[k],
                dst_ref=out_ref.at[pl.ds(my_p * m + k * mc, mc), :],
                send_sem=send_sems.at[k],
                recv_sem=recv_sems.at[k],
                device_id=(peer, my_y, my_z),
                device_id_type=pl.DeviceIdType.MESH,
            )
            r.start()
            rdmas.append(r)

        out_dma(N_CHUNKS - 2).wait()
        out_dma(N_CHUNKS - 1).wait()
        for r in rdmas:
            r.wait()

    return pl.pallas_call(
        body,
        out_shape=jax.ShapeDtypeStruct((2 * m, half), jnp.bfloat16),
        in_specs=[pl.BlockSpec(memory_space=pltpu.MemorySpace.HBM)],
        out_specs=pl.BlockSpec(memory_space=pltpu.MemorySpace.HBM),
        scratch_shapes=[
            pltpu.VMEM((2, mc, n), jnp.float32),
            pltpu.VMEM((N_CHUNKS, mc, half), jnp.bfloat16),
            pltpu.VMEM((2, mc, half), jnp.bfloat16),
            pltpu.SemaphoreType.DMA((2,)),
            pltpu.SemaphoreType.DMA((2,)),
            pltpu.SemaphoreType.DMA((N_CHUNKS,)),
            pltpu.SemaphoreType.DMA((N_CHUNKS,)),
        ],
        compiler_params=pltpu.CompilerParams(
            vmem_limit_bytes=64 * 1024 * 1024,
        ),
    )(x)


# device time: 198493 ns/iter; 1.0793x vs baseline; 1.0793x over previous
import jax
import jax.numpy as jnp
from jax import lax
from jax.experimental import pallas as pl
from jax.experimental.pallas import tpu as pltpu

N_ROUNDS = 8


def kernel(x):
    m, n = x.shape
    half = n // 2
    mc = m // N_ROUNDS
    q4 = mc // 4
    h2 = mc // 2

    def body(x_ref, out_ref, in_buf, send_buf, loc_buf,
             in_sems, out_sems, sx, rx, sy, ry, sz, rz):
        my_p = lax.axis_index("x")
        my_y = lax.axis_index("y")
        my_z = lax.axis_index("z")
        peer = 1 - my_p
        ie = 2 * my_z + my_y
        io = 2 * my_y + my_z
        qbase = peer * m

        def in_dma(c):
            return pltpu.make_async_copy(
                x_ref.at[pl.ds(c * mc, mc), :],
                in_buf.at[c % 2],
                in_sems.at[c % 2],
            )

        def out_dma(c):
            return pltpu.make_async_copy(
                loc_buf.at[c % 2],
                out_ref.at[pl.ds(my_p * m + c * mc, mc), :],
                out_sems.at[c % 2],
            )

        xds = []

        def pack_and_x(c):
            in_dma(c).wait()
            if c + 1 < N_ROUNDS:
                in_dma(c + 1).start()
            if c >= 2:
                out_dma(c - 2).wait()
                xds[c - 2].wait_send()

            @pl.when(my_p == 0)
            def _():
                loc_buf[c % 2] = in_buf[c % 2, :, :half].astype(jnp.bfloat16)
                send_buf[c % 2] = in_buf[c % 2, :, half:].astype(jnp.bfloat16)

            @pl.when(my_p == 1)
            def _():
                loc_buf[c % 2] = in_buf[c % 2, :, half:].astype(jnp.bfloat16)
                send_buf[c % 2] = in_buf[c % 2, :, :half].astype(jnp.bfloat16)

            out_dma(c).start()
            ii = ie if c % 2 == 0 else io
            xd = pltpu.make_async_remote_copy(
                src_ref=send_buf.at[c % 2, pl.ds(ii * q4, q4), :],
                dst_ref=out_ref.at[pl.ds(my_p * m + c * mc + ii * q4, q4), :],
                send_sem=sx.at[c],
                recv_sem=rx.at[c],
                device_id=(peer, my_y, my_z),
                device_id_type=pl.DeviceIdType.MESH,
            )
            xd.start()
            xds.append(xd)

        def fwd(c):
            first_z = c % 2 == 1
            ii = io if first_z else ie
            xds[c].wait_recv()
            quarter = out_ref.at[pl.ds(qbase + c * mc + ii * q4, q4), :]
            d1 = pltpu.make_async_remote_copy(
                src_ref=quarter,
                dst_ref=quarter,
                send_sem=(sz if first_z else sy).at[c],
                recv_sem=(rz if first_z else ry).at[c],
                device_id=(my_p, my_y, 1 - my_z) if first_z
                else (my_p, 1 - my_y, my_z),
                device_id_type=pl.DeviceIdType.MESH,
            )
            d1.start()
            d1.wait_recv()
            off = (my_y if first_z else my_z) * h2
            hband = out_ref.at[pl.ds(qbase + c * mc + off, h2), :]
            d2 = pltpu.make_async_remote_copy(
                src_ref=hband,
                dst_ref=hband,
                send_sem=(sy if first_z else sz).at[c],
                recv_sem=(ry if first_z else rz).at[c],
                device_id=(my_p, 1 - my_y, my_z) if first_z
                else (my_p, my_y, 1 - my_z),
                device_id_type=pl.DeviceIdType.MESH,
            )
            d2.start()
            return d1, d2

        in_dma(0).start()
        pack_and_x(0)
        d2s = []
        for c in range(N_ROUNDS):
            if c + 1 < N_ROUNDS:
                pack_and_x(c + 1)
            d2s.append(fwd(c))

        out_dma(N_ROUNDS - 2).wait()
        out_dma(N_ROUNDS - 1).wait()
        for c in range(N_ROUNDS - 2, N_ROUNDS):
            xds[c].wait_send()
        for d1, d2 in d2s:
            d1.wait_send()
            d2.wait_send()
            d2.wait_recv()

    return pl.pallas_call(
        body,
        out_shape=jax.ShapeDtypeStruct((2 * m, half), jnp.bfloat16),
        in_specs=[pl.BlockSpec(memory_space=pltpu.MemorySpace.HBM)],
        out_specs=pl.BlockSpec(memory_space=pltpu.MemorySpace.HBM),
        scratch_shapes=[
            pltpu.VMEM((2, mc, n), jnp.float32),
            pltpu.VMEM((2, mc, half), jnp.bfloat16),
            pltpu.VMEM((2, mc, half), jnp.bfloat16),
            pltpu.SemaphoreType.DMA((2,)),
            pltpu.SemaphoreType.DMA((2,)),
            pltpu.SemaphoreType.DMA((N_ROUNDS,)),
            pltpu.SemaphoreType.DMA((N_ROUNDS,)),
            pltpu.SemaphoreType.DMA((N_ROUNDS,)),
            pltpu.SemaphoreType.DMA((N_ROUNDS,)),
            pltpu.SemaphoreType.DMA((N_ROUNDS,)),
            pltpu.SemaphoreType.DMA((N_ROUNDS,)),
        ],
        compiler_params=pltpu.CompilerParams(
            vmem_limit_bytes=64 * 1024 * 1024,
        ),
    )(x)


# device time: 114665 ns/iter; 1.8684x vs baseline; 1.7311x over previous
import jax
import jax.numpy as jnp
from jax import lax
from jax.experimental import pallas as pl
from jax.experimental.pallas import tpu as pltpu

N_ROUNDS = 8


def kernel(x):
    m, n = x.shape
    half = n // 2
    mc = m // N_ROUNDS
    q4 = mc // 4

    def body(x_ref, out_ref, in_buf, send_buf, loc_buf, in_sems, out_sems,
             sx, rx, s1, r1, s2a, r2a, s2b, r2b):
        my_p = lax.axis_index("x")
        my_y = lax.axis_index("y")
        my_z = lax.axis_index("z")
        peer = 1 - my_p
        ii = 2 * my_z + my_y
        i_y = 2 * my_z + (1 - my_y)
        i_z = 2 * (1 - my_z) + my_y
        qbase = peer * m
        y_nbr = (my_p, 1 - my_y, my_z)
        z_nbr = (my_p, my_y, 1 - my_z)

        def in_dma(c):
            return pltpu.make_async_copy(
                x_ref.at[pl.ds(c * mc, mc), :],
                in_buf.at[c % 2],
                in_sems.at[c % 2],
            )

        def out_dma(c):
            return pltpu.make_async_copy(
                loc_buf.at[c % 2],
                out_ref.at[pl.ds(my_p * m + c * mc, mc), :],
                out_sems.at[c % 2],
            )

        def quarter(c, idx):
            return out_ref.at[pl.ds(qbase + c * mc + idx * q4, q4), :]

        xds, d1s, d2as, d2bs = [], [], [], []

        def pack_and_x(c):
            in_dma(c).wait()
            if c + 1 < N_ROUNDS:
                in_dma(c + 1).start()
            if c >= 2:
                out_dma(c - 2).wait()
                xds[c - 2].wait_send()

            @pl.when(my_p == 0)
            def _():
                loc_buf[c % 2] = in_buf[c % 2, :, :half].astype(jnp.bfloat16)
                send_buf[c % 2] = in_buf[c % 2, :, half:].astype(jnp.bfloat16)

            @pl.when(my_p == 1)
            def _():
                loc_buf[c % 2] = in_buf[c % 2, :, half:].astype(jnp.bfloat16)
                send_buf[c % 2] = in_buf[c % 2, :, :half].astype(jnp.bfloat16)

            out_dma(c).start()
            xd = pltpu.make_async_remote_copy(
                src_ref=send_buf.at[c % 2, pl.ds(ii * q4, q4), :],
                dst_ref=out_ref.at[pl.ds(my_p * m + c * mc + ii * q4, q4), :],
                send_sem=sx.at[c % 2],
                recv_sem=rx.at[c],
                device_id=(peer, my_y, my_z),
                device_id_type=pl.DeviceIdType.MESH,
            )
            xd.start()
            xds.append(xd)

        def fwd_own(c):
            a_nbr, b_nbr = (y_nbr, z_nbr) if c % 2 == 0 else (z_nbr, y_nbr)
            xds[c].wait_recv()
            if c >= 2:
                d1s[c - 2].wait_send()
                d2as[c - 2].wait_send()
            mine = quarter(c, ii)
            d1 = pltpu.make_async_remote_copy(
                src_ref=mine, dst_ref=mine,
                send_sem=s1.at[c % 2], recv_sem=r1.at[c],
                device_id=a_nbr, device_id_type=pl.DeviceIdType.MESH,
            )
            d1.start()
            d2a = pltpu.make_async_remote_copy(
                src_ref=mine, dst_ref=mine,
                send_sem=s2a.at[c % 2], recv_sem=r2a.at[c],
                device_id=b_nbr, device_id_type=pl.DeviceIdType.MESH,
            )
            d2a.start()
            d1s.append(d1)
            d2as.append(d2a)

        def fwd_diag(c):
            b_nbr = z_nbr if c % 2 == 0 else y_nbr
            ia = i_y if c % 2 == 0 else i_z
            d1s[c].wait_recv()
            if c >= 2:
                d2bs[c - 2].wait_send()
            q = quarter(c, ia)
            d2b = pltpu.make_async_remote_copy(
                src_ref=q, dst_ref=q,
                send_sem=s2b.at[c % 2], recv_sem=r2b.at[c],
                device_id=b_nbr, device_id_type=pl.DeviceIdType.MESH,
            )
            d2b.start()
            d2bs.append(d2b)

        in_dma(0).start()
        pack_and_x(0)
        if N_ROUNDS > 1:
            pack_and_x(1)
        for c in range(N_ROUNDS):
            if c + 2 < N_ROUNDS:
                pack_and_x(c + 2)
            fwd_own(c)
            if c >= 2:
                fwd_diag(c - 2)
        for c in range(max(N_ROUNDS - 2, 0), N_ROUNDS):
            fwd_diag(c)

        out_dma(N_ROUNDS - 2).wait()
        out_dma(N_ROUNDS - 1).wait()
        for c in range(N_ROUNDS):
            d2as[c].wait_recv()
            d2bs[c].wait_recv()
        for c in range(N_ROUNDS - 2, N_ROUNDS):
            xds[c].wait_send()
            d1s[c].wait_send()
            d2as[c].wait_send()
            d2bs[c].wait_send()

    return pl.pallas_call(
        body,
        out_shape=jax.ShapeDtypeStruct((2 * m, half), jnp.bfloat16),
        in_specs=[pl.BlockSpec(memory_space=pltpu.MemorySpace.HBM)],
        out_specs=pl.BlockSpec(memory_space=pltpu.MemorySpace.HBM),
        scratch_shapes=[
            pltpu.VMEM((2, mc, n), jnp.float32),
            pltpu.VMEM((2, mc, half), jnp.bfloat16),
            pltpu.VMEM((2, mc, half), jnp.bfloat16),
            pltpu.SemaphoreType.DMA((2,)),
            pltpu.SemaphoreType.DMA((2,)),
            pltpu.SemaphoreType.DMA((2,)),
            pltpu.SemaphoreType.DMA((N_ROUNDS,)),
            pltpu.SemaphoreType.DMA((2,)),
            pltpu.SemaphoreType.DMA((N_ROUNDS,)),
            pltpu.SemaphoreType.DMA((2,)),
            pltpu.SemaphoreType.DMA((N_ROUNDS,)),
            pltpu.SemaphoreType.DMA((2,)),
            pltpu.SemaphoreType.DMA((N_ROUNDS,)),
        ],
        compiler_params=pltpu.CompilerParams(
            vmem_limit_bytes=64 * 1024 * 1024,
        ),
    )(x)
